# Initial kernel scaffold; baseline (speedup 1.0000x reference)
import jax
import jax.numpy as jnp
from jax.experimental import pallas as pl


def kernel(q, k_sf, k_df1, k_df2, k_all_sf, k_all_df1, k_all_df2, memory):
    raise NotImplementedError("write your pallas kernel here")


# speedup vs baseline: 1.6398x; 1.6398x over previous
import jax
import jax.numpy as jnp
from jax.experimental import pallas as pl
from jax.experimental.pallas import tpu as pltpu

FEAT = 256
QS = 16384
BATCH = 1024
INV_T = 1.0 / 0.07
CW = 512
NCB = (QS + 1 + CW - 1) // CW


def _logits_body(q_ref, m1_ref, ksf_ref, kdf1_ref, kdf2_ref, out_ref, lpos_ref):
    c = pl.program_id(0)
    q = q_ref[...]
    tile = jax.lax.dot_general(
        q, m1_ref[...], (((1,), (1,)), ((), ())),
        preferred_element_type=jnp.float32,
    ) * INV_T
    out_ref[0] = tile
    out_ref[1] = tile
    out_ref[2] = tile

    @pl.when(c == 0)
    def _():
        psf = jnp.sum(q * ksf_ref[...], axis=1, keepdims=True)
        pd1 = jnp.sum(q * kdf1_ref[...], axis=1, keepdims=True)
        pd2 = jnp.sum(q * kdf2_ref[...], axis=1, keepdims=True)
        lpos_ref[...] = psf
        out_ref[0, :, 0:1] = psf * INV_T
        out_ref[1, :, 0:1] = pd1 * INV_T
        out_ref[2, :, 0:1] = pd2 * INV_T


def _update_body(ksf_ref, kdf1_ref, kdf2_ref, mem_ref, out_ref):
    i = pl.program_id(0)

    @pl.when(i == 0)
    def _():
        out_ref[...] = ksf_ref[...]

    @pl.when(i == 1)
    def _():
        out_ref[...] = kdf1_ref[...]

    @pl.when(i == 2)
    def _():
        out_ref[...] = kdf2_ref[...]

    @pl.when(i >= 3)
    def _():
        out_ref[...] = mem_ref[...]


def kernel(q, k_sf, k_df1, k_df2, k_all_sf, k_all_df1, k_all_df2, memory):
    m1 = jnp.concatenate([jnp.zeros((1, FEAT), jnp.float32), memory], axis=0)

    out3, l_pos_sf = pl.pallas_call(
        _logits_body,
        grid=(NCB,),
        in_specs=[
            pl.BlockSpec((BATCH, FEAT), lambda c: (0, 0)),
            pl.BlockSpec((CW, FEAT), lambda c: (c, 0)),
            pl.BlockSpec((BATCH, FEAT), lambda c: (0, 0)),
            pl.BlockSpec((BATCH, FEAT), lambda c: (0, 0)),
            pl.BlockSpec((BATCH, FEAT), lambda c: (0, 0)),
        ],
        out_specs=[
            pl.BlockSpec((3, BATCH, CW), lambda c: (0, 0, c)),
            pl.BlockSpec((BATCH, 1), lambda c: (0, 0)),
        ],
        out_shape=[
            jax.ShapeDtypeStruct((3, BATCH, QS + 1), jnp.float32),
            jax.ShapeDtypeStruct((BATCH, 1), jnp.float32),
        ],
    )(q, m1, k_sf, k_df1, k_df2)

    new_memory = pl.pallas_call(
        _update_body,
        grid=(16,),
        in_specs=[
            pl.BlockSpec((BATCH, FEAT), lambda i: (0, 0)),
            pl.BlockSpec((BATCH, FEAT), lambda i: (0, 0)),
            pl.BlockSpec((BATCH, FEAT), lambda i: (0, 0)),
            pl.BlockSpec((BATCH, FEAT), lambda i: (i, 0)),
        ],
        out_specs=pl.BlockSpec((BATCH, FEAT), lambda i: (i, 0)),
        out_shape=jax.ShapeDtypeStruct((QS, FEAT), jnp.float32),
    )(k_all_sf, k_all_df1, k_all_df2, memory)

    out = out3.reshape(3 * BATCH, QS + 1)
    return (out, l_pos_sf, new_memory)
